# baseline (speedup 1.0000x reference)
import functools

import jax
import jax.numpy as jnp
from jax.experimental import pallas as pl
from jax.experimental.pallas import tpu as pltpu


def _mha_body(x_ref, wqkv_ref, wpt_ref, bp_ref, o_ref,
              x_bf_ref, qkv_ref, wpt_sc_ref,
              *, num_heads, seq_len, seqs_per_block, n_strips):
    j = pl.program_id(0)
    BT, C = x_ref.shape
    hs = C // num_heads
    T = seq_len
    R = seqs_per_block * T
    scols = 3 * C // n_strips
    heads_per_strip = C // hs // (n_strips // 3)

    @pl.when(j < n_strips)
    def _():
        @pl.when(j == 0)
        def _():
            x_bf_ref[...] = x_ref[...].astype(jnp.bfloat16)

        strip = jnp.dot(x_bf_ref[...], wqkv_ref[...],
                        preferred_element_type=jnp.float32)
        qkv_ref[j] = strip.astype(jnp.bfloat16)

        wpt_sc_ref[pl.ds(j * (C // n_strips), C // n_strips), :] = wpt_ref[...]

    @pl.when(j >= n_strips)
    def _():
        i0 = (j - n_strips) * R

        row = jax.lax.broadcasted_iota(jnp.int32, (R, R), 0)
        col = jax.lax.broadcasted_iota(jnp.int32, (R, R), 1)
        keep = (col <= row) & ((row // T) == (col // T))
        neg_big = jnp.float32(-1e30)

        head_outs = []
        for h in range(num_heads):
            sq = h // heads_per_strip
            lane = (h % heads_per_strip) * hs
            q = qkv_ref[sq, pl.ds(i0, R), lane:lane + hs]
            k = qkv_ref[n_strips // 3 + sq, pl.ds(i0, R), lane:lane + hs]
            v = qkv_ref[2 * (n_strips // 3) + sq, pl.ds(i0, R), lane:lane + hs]

            s = jax.lax.dot_general(q, k,
                                    dimension_numbers=(((1,), (1,)), ((), ())),
                                    preferred_element_type=jnp.float32)
            s = jnp.where(keep, s, neg_big)
            s = s - jnp.max(s, axis=-1, keepdims=True)
            p = jnp.exp(s)
            p = p * pl.reciprocal(jnp.sum(p, axis=-1, keepdims=True),
                                  approx=True)
            head_outs.append(jnp.dot(p.astype(jnp.bfloat16), v,
                                     preferred_element_type=jnp.float32))

        cat = jnp.concatenate(head_outs, axis=-1).astype(jnp.bfloat16)
        proj = jnp.dot(cat, wpt_sc_ref[...], preferred_element_type=jnp.float32)
        o_ref[...] = proj + bp_ref[...].astype(jnp.float32)


@functools.partial(jax.jit, static_argnames=("num_heads", "seqs_per_block",
                                             "n_strips"))
def _mha(x, wqkv_bf, wpt_bf, bp_f32, *, num_heads, seqs_per_block, n_strips):
    B, T, C = x.shape
    BT = B * T
    R = seqs_per_block * T
    n_row_steps = B // seqs_per_block
    scols = 3 * C // n_strips
    n_steps = n_strips + n_row_steps

    body = functools.partial(_mha_body, num_heads=num_heads, seq_len=T,
                             seqs_per_block=seqs_per_block, n_strips=n_strips)
    out = pl.pallas_call(
        body,
        out_shape=jax.ShapeDtypeStruct((BT, C), jnp.float32),
        grid=(n_steps,),
        in_specs=[
            pl.BlockSpec((BT, C), lambda j: (0, 0)),
            pl.BlockSpec((C, scols),
                         lambda j: (0, jnp.minimum(j, n_strips - 1))),
            pl.BlockSpec((C // n_strips, C),
                         lambda j: (jnp.minimum(j, n_strips - 1), 0)),
            pl.BlockSpec((1, C), lambda j: (0, 0)),
        ],
        out_specs=pl.BlockSpec(
            (R, C), lambda j: (jnp.maximum(j - n_strips, 0), 0)),
        scratch_shapes=[
            pltpu.VMEM((BT, C), jnp.bfloat16),
            pltpu.VMEM((n_strips, BT, scols), jnp.bfloat16),
            pltpu.VMEM((C, C), jnp.bfloat16),
        ],
        compiler_params=pltpu.CompilerParams(
            dimension_semantics=("arbitrary",)),
        name="mha_stream",
    )(x.reshape(BT, C), wqkv_bf, wpt_bf, bp_f32)

    return out.reshape(B, T, C)


def kernel(x, wqkv_bf, wpt_bf, bp_f32):
    return _mha(x, wqkv_bf, wpt_bf, bp_f32, num_heads=12, seqs_per_block=2,
                n_strips=6)
